# baseline (speedup 1.0000x reference)
import math

import jax
import jax.numpy as jnp
from jax.experimental import pallas as pl
from jax.experimental.pallas import tpu as pltpu

B, T, H, D_H = 1, 2048, 16, 128
NP = 8
S = B * T * H
TILE = 2048


def _fused_body(x_ref, pt_ref, gate_ref, w1_ref, w2_ref,
                out_ref, logits_t_ref, mask_t_ref):
    x = x_ref[...]
    inv = 1.0 / math.sqrt(D_H)
    logits = jnp.dot(x, pt_ref[...],
                     preferred_element_type=jnp.float32) * inv - gate_ref[...]
    mask = jnp.maximum(logits, 0.0)
    logits_t_ref[...] = logits.T
    mask_t_ref[...] = mask.T
    w = jnp.where(mask > 1e-6, mask, 0.0).astype(jnp.bfloat16)

    xb = x.astype(jnp.bfloat16)
    h1 = jnp.dot(xb, w1_ref[...],
                 preferred_element_type=jnp.float32).astype(jnp.bfloat16)
    a = h1 * (0.5 * jnp.tanh(h1 * 0.5) + 0.5)
    parts = []
    for p in range(NP):
        parts.append(a[:, p * D_H:(p + 1) * D_H] * w[:, p:p + 1])
    a_scaled = jnp.concatenate(parts, axis=1)
    out_ref[...] = jnp.dot(a_scaled, w2_ref[...],
                           preferred_element_type=jnp.float32)


def kernel(x_proj, proto, gate, W1, W2):
    x_flat = x_proj.reshape(S, D_H)
    proto_t = proto.T
    gate2 = gate.reshape(1, NP)
    w1cat = jnp.transpose(W1, (2, 0, 1)).reshape(D_H, NP * D_H).astype(jnp.bfloat16)
    w2cat = jnp.transpose(W2, (0, 2, 1)).reshape(NP * D_H, D_H).astype(jnp.bfloat16)

    grid = (S // TILE,)
    out, logits_t, mask_t = pl.pallas_call(
        _fused_body,
        grid=grid,
        in_specs=[
            pl.BlockSpec((TILE, D_H), lambda i: (i, 0)),
            pl.BlockSpec((D_H, NP), lambda i: (0, 0)),
            pl.BlockSpec((1, NP), lambda i: (0, 0)),
            pl.BlockSpec((D_H, NP * D_H), lambda i: (0, 0)),
            pl.BlockSpec((NP * D_H, D_H), lambda i: (0, 0)),
        ],
        out_specs=[
            pl.BlockSpec((TILE, D_H), lambda i: (i, 0)),
            pl.BlockSpec((NP, TILE), lambda i: (0, i)),
            pl.BlockSpec((NP, TILE), lambda i: (0, i)),
        ],
        out_shape=[
            jax.ShapeDtypeStruct((S, D_H), jnp.float32),
            jax.ShapeDtypeStruct((NP, S), jnp.float32),
            jax.ShapeDtypeStruct((NP, S), jnp.float32),
        ],
        compiler_params=pltpu.CompilerParams(
            dimension_semantics=("parallel",)),
    )(x_flat, proto_t, gate2, w1cat, w2cat)

    logits = logits_t.T
    mask = mask_t.T
    active_mask = mask > 1e-6
    return (out.reshape(B, T, H, D_H),
            logits.reshape(B, T, H, NP),
            mask.reshape(B, T, H, NP),
            active_mask)


# speedup vs baseline: 1.0204x; 1.0204x over previous
import math

import jax
import jax.numpy as jnp
from jax.experimental import pallas as pl
from jax.experimental.pallas import tpu as pltpu

B, T, H, D_H = 1, 2048, 16, 128
NP = 8
S = B * T * H
TILE = 4096


def _fused_body(x_ref, pt_ref, gate_ref, w1_ref, w2_ref,
                out_ref, logits_t_ref, mask_t_ref):
    x = x_ref[...]
    inv = 1.0 / math.sqrt(D_H)
    logits = jnp.dot(x, pt_ref[...],
                     preferred_element_type=jnp.float32) * inv - gate_ref[...]
    mask = jnp.maximum(logits, 0.0)
    logits_t_ref[...] = logits.T
    mask_t_ref[...] = mask.T
    w = jnp.where(mask > 1e-6, mask, 0.0).astype(jnp.bfloat16)

    xb = x.astype(jnp.bfloat16)
    h1 = jnp.dot(xb, w1_ref[...],
                 preferred_element_type=jnp.float32).astype(jnp.bfloat16)
    a = h1 * (0.5 * jnp.tanh(h1 * 0.5) + 0.5)
    parts = []
    for p in range(NP):
        parts.append(a[:, p * D_H:(p + 1) * D_H] * w[:, p:p + 1])
    a_scaled = jnp.concatenate(parts, axis=1)
    out_ref[...] = jnp.dot(a_scaled, w2_ref[...],
                           preferred_element_type=jnp.float32)


def kernel(x_proj, proto, gate, W1, W2):
    x_flat = x_proj.reshape(S, D_H)
    proto_t = proto.T
    gate2 = gate.reshape(1, NP)
    w1cat = jnp.transpose(W1, (2, 0, 1)).reshape(D_H, NP * D_H).astype(jnp.bfloat16)
    w2cat = jnp.transpose(W2, (0, 2, 1)).reshape(NP * D_H, D_H).astype(jnp.bfloat16)

    grid = (S // TILE,)
    out, logits_t, mask_t = pl.pallas_call(
        _fused_body,
        grid=grid,
        in_specs=[
            pl.BlockSpec((TILE, D_H), lambda i: (i, 0)),
            pl.BlockSpec((D_H, NP), lambda i: (0, 0)),
            pl.BlockSpec((1, NP), lambda i: (0, 0)),
            pl.BlockSpec((D_H, NP * D_H), lambda i: (0, 0)),
            pl.BlockSpec((NP * D_H, D_H), lambda i: (0, 0)),
        ],
        out_specs=[
            pl.BlockSpec((TILE, D_H), lambda i: (i, 0)),
            pl.BlockSpec((NP, TILE), lambda i: (0, i)),
            pl.BlockSpec((NP, TILE), lambda i: (0, i)),
        ],
        out_shape=[
            jax.ShapeDtypeStruct((S, D_H), jnp.float32),
            jax.ShapeDtypeStruct((NP, S), jnp.float32),
            jax.ShapeDtypeStruct((NP, S), jnp.float32),
        ],
        compiler_params=pltpu.CompilerParams(
            dimension_semantics=("parallel",)),
    )(x_flat, proto_t, gate2, w1cat, w2cat)

    logits = logits_t.T
    mask = mask_t.T
    active_mask = mask > 1e-6
    return (out.reshape(B, T, H, D_H),
            logits.reshape(B, T, H, NP),
            mask.reshape(B, T, H, NP),
            active_mask)


# speedup vs baseline: 1.0227x; 1.0022x over previous
import math

import jax
import jax.numpy as jnp
from jax.experimental import pallas as pl
from jax.experimental.pallas import tpu as pltpu

B, T, H, D_H = 1, 2048, 16, 128
NP = 8
S = B * T * H
TILE = 4096


def _fused_body(x_ref, pt_ref, gate_ref, w1_ref, w2_ref,
                out_ref, logits_t_ref, mask_t_ref):
    x = x_ref[...]
    inv = 1.0 / math.sqrt(D_H)
    logits = jnp.dot(x, pt_ref[...],
                     preferred_element_type=jnp.float32) * inv - gate_ref[...]
    mask = jnp.maximum(logits, 0.0)
    logits_t_ref[...] = logits.T
    mask_t_ref[...] = mask.T
    w = jnp.where(mask > 1e-6, mask, 0.0).astype(jnp.bfloat16)

    xb = x.astype(jnp.bfloat16)
    h2 = jnp.dot(xb, w1_ref[...],
                 preferred_element_type=jnp.float32).astype(jnp.bfloat16)
    a = h2 * jnp.tanh(h2) + h2
    parts = []
    for p in range(NP):
        parts.append(a[:, p * D_H:(p + 1) * D_H] * w[:, p:p + 1])
    a_scaled = jnp.concatenate(parts, axis=1)
    out_ref[...] = jnp.dot(a_scaled, w2_ref[...],
                           preferred_element_type=jnp.float32)


def kernel(x_proj, proto, gate, W1, W2):
    x_flat = x_proj.reshape(S, D_H)
    proto_t = proto.T
    gate2 = gate.reshape(1, NP)
    w1cat = (0.5 * jnp.transpose(W1, (2, 0, 1)).reshape(D_H, NP * D_H)).astype(jnp.bfloat16)
    w2cat = jnp.transpose(W2, (0, 2, 1)).reshape(NP * D_H, D_H).astype(jnp.bfloat16)

    grid = (S // TILE,)
    out, logits_t, mask_t = pl.pallas_call(
        _fused_body,
        grid=grid,
        in_specs=[
            pl.BlockSpec((TILE, D_H), lambda i: (i, 0)),
            pl.BlockSpec((D_H, NP), lambda i: (0, 0)),
            pl.BlockSpec((1, NP), lambda i: (0, 0)),
            pl.BlockSpec((D_H, NP * D_H), lambda i: (0, 0)),
            pl.BlockSpec((NP * D_H, D_H), lambda i: (0, 0)),
        ],
        out_specs=[
            pl.BlockSpec((TILE, D_H), lambda i: (i, 0)),
            pl.BlockSpec((NP, TILE), lambda i: (0, i)),
            pl.BlockSpec((NP, TILE), lambda i: (0, i)),
        ],
        out_shape=[
            jax.ShapeDtypeStruct((S, D_H), jnp.float32),
            jax.ShapeDtypeStruct((NP, S), jnp.float32),
            jax.ShapeDtypeStruct((NP, S), jnp.float32),
        ],
        compiler_params=pltpu.CompilerParams(
            dimension_semantics=("parallel",)),
    )(x_flat, proto_t, gate2, w1cat, w2cat)

    logits = logits_t.T
    mask = mask_t.T
    active_mask = mask > 1e-6
    return (out.reshape(B, T, H, D_H),
            logits.reshape(B, T, H, NP),
            mask.reshape(B, T, H, NP),
            active_mask)


# speedup vs baseline: 1.0232x; 1.0005x over previous
import math

import jax
import jax.numpy as jnp
from jax.experimental import pallas as pl
from jax.experimental.pallas import tpu as pltpu

B, T, H, D_H = 1, 2048, 16, 128
NP = 8
S = B * T * H
TILE = 8192


def _fused_body(x_ref, pt_ref, gate_ref, w1_ref, w2_ref,
                out_ref, logits_t_ref, mask_t_ref):
    x = x_ref[...]
    inv = 1.0 / math.sqrt(D_H)
    logits = jnp.dot(x, pt_ref[...],
                     preferred_element_type=jnp.float32) * inv - gate_ref[...]
    mask = jnp.maximum(logits, 0.0)
    logits_t_ref[...] = logits.T
    mask_t_ref[...] = mask.T
    w = jnp.where(mask > 1e-6, mask, 0.0).astype(jnp.bfloat16)

    xb = x.astype(jnp.bfloat16)
    h2 = jnp.dot(xb, w1_ref[...],
                 preferred_element_type=jnp.float32).astype(jnp.bfloat16)
    a = h2 * jnp.tanh(h2) + h2
    parts = []
    for p in range(NP):
        parts.append(a[:, p * D_H:(p + 1) * D_H] * w[:, p:p + 1])
    a_scaled = jnp.concatenate(parts, axis=1)
    out_ref[...] = jnp.dot(a_scaled, w2_ref[...],
                           preferred_element_type=jnp.float32)


def kernel(x_proj, proto, gate, W1, W2):
    x_flat = x_proj.reshape(S, D_H)
    proto_t = proto.T
    gate2 = gate.reshape(1, NP)
    w1cat = (0.5 * jnp.transpose(W1, (2, 0, 1)).reshape(D_H, NP * D_H)).astype(jnp.bfloat16)
    w2cat = jnp.transpose(W2, (0, 2, 1)).reshape(NP * D_H, D_H).astype(jnp.bfloat16)

    grid = (S // TILE,)
    out, logits_t, mask_t = pl.pallas_call(
        _fused_body,
        grid=grid,
        in_specs=[
            pl.BlockSpec((TILE, D_H), lambda i: (i, 0)),
            pl.BlockSpec((D_H, NP), lambda i: (0, 0)),
            pl.BlockSpec((1, NP), lambda i: (0, 0)),
            pl.BlockSpec((D_H, NP * D_H), lambda i: (0, 0)),
            pl.BlockSpec((NP * D_H, D_H), lambda i: (0, 0)),
        ],
        out_specs=[
            pl.BlockSpec((TILE, D_H), lambda i: (i, 0)),
            pl.BlockSpec((NP, TILE), lambda i: (0, i)),
            pl.BlockSpec((NP, TILE), lambda i: (0, i)),
        ],
        out_shape=[
            jax.ShapeDtypeStruct((S, D_H), jnp.float32),
            jax.ShapeDtypeStruct((NP, S), jnp.float32),
            jax.ShapeDtypeStruct((NP, S), jnp.float32),
        ],
        compiler_params=pltpu.CompilerParams(
            dimension_semantics=("parallel",)),
    )(x_flat, proto_t, gate2, w1cat, w2cat)

    logits = logits_t.T
    mask = mask_t.T
    active_mask = mask > 1e-6
    return (out.reshape(B, T, H, D_H),
            logits.reshape(B, T, H, NP),
            mask.reshape(B, T, H, NP),
            active_mask)
